# baseline (speedup 1.0000x reference)
import functools

import jax
import jax.numpy as jnp
from jax import lax
from jax.experimental import pallas as pl
from jax.experimental.pallas import tpu as pltpu
from jax.experimental.pallas import tpu_sc as plsc

_B = 16384
_F = 26
_V = 100000
_D = 32


_VB = 4096
_NVB = (_V + _VB - 1) // _VB


def _tpose_body(x_ref, y_ref):
    y_ref[...] = jnp.transpose(x_ref[...], (0, 2, 1))


_tpose = pl.pallas_call(
    _tpose_body,
    grid=(_F, _NVB),
    in_specs=[pl.BlockSpec((1, _D, _VB), lambda f, v: (f, 0, v))],
    out_specs=pl.BlockSpec((1, _VB, _D), lambda f, v: (f, v, 0)),
    out_shape=jax.ShapeDtypeStruct((_F, _V, _D), jnp.float32),
    compiler_params=pltpu.CompilerParams(
        dimension_semantics=("parallel", "parallel")),
)


_info = plsc.get_sparse_core_info()
_NC = _info.num_cores
_NS = _info.num_subcores
_L = _info.num_lanes
_NW = _NC * _NS
_BW = _B // _NW
_BC = 64
_NCK = _BW // _BC
_CH = 128
_NG = _BC * _F // _CH
_NROW = _BC * _F

_mesh = plsc.VectorSubcoreMesh(core_axis_name="c", subcore_axis_name="s")


@functools.partial(
    pl.kernel,
    mesh=_mesh,
    out_type=jax.ShapeDtypeStruct((_B, _D), jnp.float32),
    compiler_params=pltpu.CompilerParams(use_tc_tiling_on_sc=False),
    scratch_types=[
        pltpu.VMEM((_NG, _CH), jnp.int32),
        pltpu.VMEM((_NG, _CH), jnp.int32),
        pltpu.VMEM((_NG, _CH), jnp.int32),
        pltpu.VMEM((_NROW, _D), jnp.float32),
        pltpu.VMEM((_NROW, _D), jnp.float32),
        pltpu.VMEM((_BC, _D), jnp.float32),
        pltpu.VMEM((_BC, _D), jnp.float32),
        pltpu.SemaphoreType.DMA,
        pltpu.SemaphoreType.DMA,
        pltpu.SemaphoreType.DMA,
        pltpu.SemaphoreType.DMA,
    ],
)
def _emb_sum(idx_hbm, tab_hbm, out_hbm,
             off_v, idx0, idx1, buf0, buf1, ost0, ost1,
             gsem0, gsem1, osem0, osem1):
    wid = lax.axis_index("s") * _NC + lax.axis_index("c")
    row0 = wid * (_BW * _F // _CH)

    for r in range(_NG):
        for s8 in range(_CH // _L):
            p = (r * _CH + s8 * _L) % _F
            fvec = p + lax.iota(jnp.int32, _L)
            fvec = jnp.where(fvec >= _F, fvec - _F, fvec)
            off_v[r, pl.ds(s8 * _L, _L)] = fvec * _V

    idxs = (idx0, idx1)
    bufs = (buf0, buf1)
    osts = (ost0, ost1)
    gsems = (gsem0, gsem1)
    osems = (osem0, osem1)

    def _fetch(c, s):
        idxv = idxs[s]
        pltpu.sync_copy(idx_hbm.at[pl.ds(row0 + c * _NG, _NG), :], idxv)
        for r in range(_NG):
            for s8 in range(_CH // _L):
                sl = pl.ds(s8 * _L, _L)
                idxv[r, sl] += off_v[r, sl]
        return [
            pltpu.async_copy(
                tab_hbm.at[idxv.at[r]],
                bufs[s].at[pl.ds(r * _CH, _CH)],
                gsems[s])
            for r in range(_NG)
        ]

    cps = [None, None]
    ocps = [None, None]
    cps[0] = _fetch(0, 0)
    for c in range(_NCK):
        s = c & 1
        if c + 1 < _NCK:
            cps[s ^ 1] = _fetch(c + 1, s ^ 1)
        for cp in cps[s]:
            cp.wait()
        if ocps[s] is not None:
            ocps[s].wait()
        buf = bufs[s]
        ost = osts[s]

        def _row(bl, _, buf=buf, ost=ost):
            r0 = bl * _F
            for half in range(2):
                sl = pl.ds(half * _L, _L)
                vals = [buf[r0 + t, sl] for t in range(_F)]
                while len(vals) > 1:
                    vals = [a + b for a, b in zip(vals[::2], vals[1::2])] + (
                        [vals[-1]] if len(vals) & 1 else [])
                ost[bl, sl] = vals[0]
            return 0
        lax.fori_loop(0, _BC, _row, 0)
        ocps[s] = pltpu.async_copy(
            ost, out_hbm.at[pl.ds((wid * _NCK + c) * _BC, _BC)], osems[s])
    for oc in ocps:
        if oc is not None:
            oc.wait()


def kernel(inputs, tables):
    xv = jnp.transpose(tables, (0, 2, 1))
    tab = _tpose(xv).reshape(_F * _V, _D)
    idx = inputs.reshape(_B * _F // _CH, _CH).astype(jnp.int32)
    return _emb_sum(idx, tab)


# speedup vs baseline: 1.5303x; 1.5303x over previous
import functools

import jax
import jax.numpy as jnp
from jax import lax
from jax.experimental import pallas as pl
from jax.experimental.pallas import tpu as pltpu
from jax.experimental.pallas import tpu_sc as plsc

_B = 16384
_F = 26
_V = 100000
_D = 32


_SPAN = 512
_NSP = (_V + _SPAN - 1) // _SPAN
_RPF = _NSP * 128


_SPG = 4
_NTB = _NSP // _SPG


def _tpose_body(x_ref, y_ref):
    for ts in range(_SPG):
        for a in range(4):
            lo = ts * _SPAN + a * 128
            y_ref[0, ts * 128:(ts + 1) * 128, a * _D:(a + 1) * _D] = (
                jnp.transpose(x_ref[0, :, lo:lo + 128], (1, 0)))


_tpose = pl.pallas_call(
    _tpose_body,
    grid=(_F, _NTB),
    in_specs=[pl.BlockSpec((1, _D, _SPG * _SPAN), lambda f, t: (f, 0, t))],
    out_specs=pl.BlockSpec((1, _SPG * 128, 4 * _D), lambda f, t: (f, t, 0)),
    out_shape=jax.ShapeDtypeStruct((_F, _RPF, 4 * _D), jnp.float32),
    compiler_params=pltpu.CompilerParams(
        dimension_semantics=("parallel", "parallel")),
)


_info = plsc.get_sparse_core_info()
_NC = _info.num_cores
_NS = _info.num_subcores
_L = _info.num_lanes
_NW = _NC * _NS
_BW = _B // _NW
_BC = 64
_NCK = _BW // _BC
_CH = 128
_NG = _BC * _F // _CH
_NROW = _BC * _F

_mesh = plsc.VectorSubcoreMesh(core_axis_name="c", subcore_axis_name="s")


@functools.partial(
    pl.kernel,
    mesh=_mesh,
    out_type=jax.ShapeDtypeStruct((_B, _D), jnp.float32),
    compiler_params=pltpu.CompilerParams(use_tc_tiling_on_sc=False),
    scratch_types=[
        pltpu.VMEM((_NG, _CH), jnp.int32),
        pltpu.VMEM((_NG, _CH), jnp.int32),
        pltpu.VMEM((_NG, _CH), jnp.int32),
        pltpu.VMEM((_NROW, _D), jnp.float32),
        pltpu.VMEM((_NROW, _D), jnp.float32),
        pltpu.VMEM((_BC, _D), jnp.float32),
        pltpu.VMEM((_BC, _D), jnp.float32),
        pltpu.SemaphoreType.DMA,
        pltpu.SemaphoreType.DMA,
        pltpu.SemaphoreType.DMA,
        pltpu.SemaphoreType.DMA,
    ],
)
def _emb_sum(idx_hbm, tab_hbm, out_hbm,
             off_v, idx0, idx1, buf0, buf1, ost0, ost1,
             gsem0, gsem1, osem0, osem1):
    wid = lax.axis_index("s") * _NC + lax.axis_index("c")
    row0 = wid * (_BW * _F // _CH)

    for r in range(_NG):
        for s8 in range(_CH // _L):
            p = (r * _CH + s8 * _L) % _F
            fvec = p + lax.iota(jnp.int32, _L)
            fvec = jnp.where(fvec >= _F, fvec - _F, fvec)
            off_v[r, pl.ds(s8 * _L, _L)] = fvec * (4 * _RPF)

    idxs = (idx0, idx1)
    bufs = (buf0, buf1)
    osts = (ost0, ost1)
    gsems = (gsem0, gsem1)
    osems = (osem0, osem1)

    def _fetch(c, s):
        idxv = idxs[s]
        pltpu.sync_copy(idx_hbm.at[pl.ds(row0 + c * _NG, _NG), :], idxv)
        for r in range(_NG):
            for s8 in range(_CH // _L):
                sl = pl.ds(s8 * _L, _L)
                v = idxv[r, sl]
                idxv[r, sl] = (off_v[r, sl] + ((v >> 9) << 9)
                               + ((v & 127) << 2) + ((v >> 7) & 3))
        return [
            pltpu.async_copy(
                tab_hbm.at[idxv.at[r]],
                bufs[s].at[pl.ds(r * _CH, _CH)],
                gsems[s])
            for r in range(_NG)
        ]

    cps = [None, None]
    ocps = [None, None]
    cps[0] = _fetch(0, 0)
    for c in range(_NCK):
        s = c & 1
        if c + 1 < _NCK:
            cps[s ^ 1] = _fetch(c + 1, s ^ 1)
        for cp in cps[s]:
            cp.wait()
        if ocps[s] is not None:
            ocps[s].wait()
        buf = bufs[s]
        ost = osts[s]

        def _row(bl, _, buf=buf, ost=ost):
            r0 = bl * _F
            for half in range(2):
                sl = pl.ds(half * _L, _L)
                vals = [buf[r0 + t, sl] for t in range(_F)]
                while len(vals) > 1:
                    vals = [a + b for a, b in zip(vals[::2], vals[1::2])] + (
                        [vals[-1]] if len(vals) & 1 else [])
                ost[bl, sl] = vals[0]
            return 0
        lax.fori_loop(0, _BC, _row, 0)
        ocps[s] = pltpu.async_copy(
            ost, out_hbm.at[pl.ds((wid * _NCK + c) * _BC, _BC)], osems[s])
    for oc in ocps:
        if oc is not None:
            oc.wait()


def kernel(inputs, tables):
    xv = jnp.transpose(tables, (0, 2, 1))
    tab = _tpose(xv).reshape(_F * _RPF * 4, _D)
    idx = inputs.reshape(_B * _F // _CH, _CH).astype(jnp.int32)
    return _emb_sum(idx, tab)


# speedup vs baseline: 3.8564x; 2.5200x over previous
import functools

import jax
import jax.numpy as jnp
from jax import lax
from jax.experimental import pallas as pl
from jax.experimental.pallas import tpu as pltpu
from jax.experimental.pallas import tpu_sc as plsc

_B = 16384
_F = 26
_V = 100000
_D = 32


_NGRP = (_F + 3) // 4
_VP = 100096
_VBK = 2048
_NTB = (_VP + _VBK - 1) // _VBK


def _tpose_body(x_ref, y_ref):
    for c in range(_VBK // 128):
        blk = x_ref[:, :, c * 128:(c + 1) * 128].reshape(128, 128)
        y_ref[0, c * 128:(c + 1) * 128, :] = jnp.transpose(blk, (1, 0))


_tpose = pl.pallas_call(
    _tpose_body,
    grid=(_NGRP, _NTB),
    in_specs=[pl.BlockSpec((4, _D, _VBK), lambda g, t: (g, 0, t))],
    out_specs=pl.BlockSpec((1, _VBK, 4 * _D), lambda g, t: (g, t, 0)),
    out_shape=jax.ShapeDtypeStruct((_NGRP, _VP, 4 * _D), jnp.float32),
    compiler_params=pltpu.CompilerParams(
        dimension_semantics=("parallel", "parallel")),
)


_info = plsc.get_sparse_core_info()
_NC = _info.num_cores
_NS = _info.num_subcores
_L = _info.num_lanes
_NW = _NC * _NS
_BW = _B // _NW
_BC = 64
_NCK = _BW // _BC
_CH = 128
_NG = _BC * _F // _CH
_NROW = _BC * _F

_mesh = plsc.VectorSubcoreMesh(core_axis_name="c", subcore_axis_name="s")


@functools.partial(
    pl.kernel,
    mesh=_mesh,
    out_type=jax.ShapeDtypeStruct((_B, _D), jnp.float32),
    compiler_params=pltpu.CompilerParams(use_tc_tiling_on_sc=False),
    scratch_types=[
        pltpu.VMEM((_NG, _CH), jnp.int32),
        pltpu.VMEM((_NG, _CH), jnp.int32),
        pltpu.VMEM((_NG, _CH), jnp.int32),
        pltpu.VMEM((_NROW, _D), jnp.float32),
        pltpu.VMEM((_NROW, _D), jnp.float32),
        pltpu.VMEM((_BC, _D), jnp.float32),
        pltpu.VMEM((_BC, _D), jnp.float32),
        pltpu.SemaphoreType.DMA,
        pltpu.SemaphoreType.DMA,
        pltpu.SemaphoreType.DMA,
        pltpu.SemaphoreType.DMA,
    ],
)
def _emb_sum(idx_hbm, tab_hbm, out_hbm,
             off_v, idx0, idx1, buf0, buf1, ost0, ost1,
             gsem0, gsem1, osem0, osem1):
    wid = lax.axis_index("s") * _NC + lax.axis_index("c")
    row0 = wid * (_BW * _F // _CH)

    for r in range(_NG):
        for s8 in range(_CH // _L):
            p = (r * _CH + s8 * _L) % _F
            fvec = p + lax.iota(jnp.int32, _L)
            fvec = jnp.where(fvec >= _F, fvec - _F, fvec)
            off_v[r, pl.ds(s8 * _L, _L)] = (
                (fvec >> 2) * (4 * _VP) + (fvec & 3))

    idxs = (idx0, idx1)
    bufs = (buf0, buf1)
    osts = (ost0, ost1)
    gsems = (gsem0, gsem1)
    osems = (osem0, osem1)

    def _fetch(c, s):
        idxv = idxs[s]
        pltpu.sync_copy(idx_hbm.at[pl.ds(row0 + c * _NG, _NG), :], idxv)
        for r in range(_NG):
            for s8 in range(_CH // _L):
                sl = pl.ds(s8 * _L, _L)
                idxv[r, sl] = off_v[r, sl] + (idxv[r, sl] << 2)
        return [
            pltpu.async_copy(
                tab_hbm.at[idxv.at[r]],
                bufs[s].at[pl.ds(r * _CH, _CH)],
                gsems[s])
            for r in range(_NG)
        ]

    cps = [None, None]
    ocps = [None, None]
    cps[0] = _fetch(0, 0)
    for c in range(_NCK):
        s = c & 1
        if c + 1 < _NCK:
            cps[s ^ 1] = _fetch(c + 1, s ^ 1)
        for cp in cps[s]:
            cp.wait()
        if ocps[s] is not None:
            ocps[s].wait()
        buf = bufs[s]
        ost = osts[s]

        def _row(bl, _, buf=buf, ost=ost):
            r0 = bl * _F
            for half in range(2):
                sl = pl.ds(half * _L, _L)
                vals = [buf[r0 + t, sl] for t in range(_F)]
                while len(vals) > 1:
                    vals = [a + b for a, b in zip(vals[::2], vals[1::2])] + (
                        [vals[-1]] if len(vals) & 1 else [])
                ost[bl, sl] = vals[0]
            return 0
        lax.fori_loop(0, _BC, _row, 0)
        ocps[s] = pltpu.async_copy(
            ost, out_hbm.at[pl.ds((wid * _NCK + c) * _BC, _BC)], osems[s])
    for oc in ocps:
        if oc is not None:
            oc.wait()


def kernel(inputs, tables):
    xv = jnp.transpose(tables, (0, 2, 1))
    tab = _tpose(xv).reshape(_NGRP * _VP * 4, _D)
    idx = inputs.reshape(_B * _F // _CH, _CH).astype(jnp.int32)
    return _emb_sum(idx, tab)
